# Initial kernel scaffold; baseline (speedup 1.0000x reference)
import jax
import jax.numpy as jnp
from jax.experimental import pallas as pl


def kernel(image_features, text_features, ln_text_w, ln_text_b, ln_local_w, ln_local_b, W1, b1, W2, b2):
    raise NotImplementedError("write your pallas kernel here")


# speedup vs baseline: 77.2431x; 77.2431x over previous
import jax
import jax.numpy as jnp
from jax.experimental import pallas as pl
from jax.experimental.pallas import tpu as pltpu

F2F_THRD = 0.98
F2T_THRD = -1.0
MAX_FRAME_NUM = 32
N = 1000
EMBED = 2560
HID = 512
BF = 40


def _pool_ln_kernel(x_ref, w_ref, b_ref, o_ref):
    x = x_ref[...]
    pooled = jnp.mean(x, axis=1)
    mu = jnp.mean(pooled, axis=1, keepdims=True)
    d = pooled - mu
    var = jnp.mean(d * d, axis=1, keepdims=True)
    o_ref[...] = d * jax.lax.rsqrt(var + 1e-5) * w_ref[...] + b_ref[...]


def _main_kernel(p_ref, t_ref, lw_ref, lb_ref, W1_ref, b1_ref, w2_ref, b2_ref,
                 sel_ref, g_ref, f2f_ref, vis_ref):
    t = t_ref[...]
    mu_t = jnp.mean(t, axis=1, keepdims=True)
    dt = t - mu_t
    var_t = jnp.mean(dt * dt, axis=1, keepdims=True)
    tn = dt * jax.lax.rsqrt(var_t + 1e-5) * lw_ref[...] + lb_ref[...]
    tvec = jnp.dot(tn, W1_ref[:EMBED, :],
                   preferred_element_type=jnp.float32)

    p = p_ref[...]
    h = jnp.dot(p, W1_ref[EMBED:, :],
                preferred_element_type=jnp.float32) + tvec + b1_ref[...]
    h = jnp.maximum(h, 0.0)
    logits = jax.lax.dot_general(
        w2_ref[...], h, (((1,), (1,)), ((), ())),
        preferred_element_type=jnp.float32) + b2_ref[...]
    gates = jax.nn.sigmoid(logits)
    g_ref[...] = gates

    nrm = jnp.sqrt(jnp.sum(p * p, axis=1, keepdims=True))
    pn = p / jnp.maximum(nrm, 1e-8)
    f2f_ref[...] = jax.lax.dot_general(
        pn, pn, (((1,), (1,)), ((), ())),
        preferred_element_type=jnp.float32)

    idx = jax.lax.broadcasted_iota(jnp.int32, (1, N), 1)
    vis_ref[...] = jnp.zeros((1, N), jnp.float32)
    sel_ref[...] = jnp.zeros((1, N), jnp.int32)

    def body(_, cnt):
        g = g_ref[...]
        v = vis_ref[...]
        masked = jnp.where(v > 0.0, -jnp.inf, g)
        m = jnp.max(masked)
        cur = jnp.min(jnp.where(masked == m, idx, N))
        active = (m >= F2T_THRD) & (cnt < MAX_FRAME_NUM)
        row = f2f_ref[pl.ds(cur, 1), :]
        nv = jnp.where((row > F2F_THRD) | (idx == cur), 1.0, v)
        vis_ref[...] = jnp.where(active, nv, v)
        sel_ref[...] = jnp.where(active & (idx == cur), 1, sel_ref[...])
        return cnt + active.astype(jnp.int32)

    jax.lax.fori_loop(0, MAX_FRAME_NUM, body, jnp.int32(0))


def kernel(image_features, text_features, ln_text_w, ln_text_b,
           ln_local_w, ln_local_b, W1, b1, W2, b2):
    n_blocks = N // BF
    pooled = pl.pallas_call(
        _pool_ln_kernel,
        grid=(n_blocks,),
        in_specs=[
            pl.BlockSpec((BF, 16, EMBED), lambda i: (i, 0, 0)),
            pl.BlockSpec((1, EMBED), lambda i: (0, 0)),
            pl.BlockSpec((1, EMBED), lambda i: (0, 0)),
        ],
        out_specs=pl.BlockSpec((BF, EMBED), lambda i: (i, 0)),
        out_shape=jax.ShapeDtypeStruct((N, EMBED), jnp.float32),
    )(image_features, ln_local_w.reshape(1, EMBED), ln_local_b.reshape(1, EMBED))

    sel, gates = pl.pallas_call(
        _main_kernel,
        out_shape=[
            jax.ShapeDtypeStruct((1, N), jnp.int32),
            jax.ShapeDtypeStruct((1, N), jnp.float32),
        ],
        scratch_shapes=[pltpu.VMEM((N, N), jnp.float32),
                        pltpu.VMEM((1, N), jnp.float32)],
    )(pooled, text_features,
      ln_text_w.reshape(1, EMBED), ln_text_b.reshape(1, EMBED),
      W1, b1.reshape(1, HID), W2.reshape(1, HID), b2.reshape(1, 1))

    return (sel[0], gates[0])


# speedup vs baseline: 78.6991x; 1.0188x over previous
import jax
import jax.numpy as jnp
from jax.experimental import pallas as pl
from jax.experimental.pallas import tpu as pltpu

F2F_THRD = 0.98
F2T_THRD = -1.0
MAX_FRAME_NUM = 32
N = 1000
EMBED = 2560
HID = 512
NP = 16
BF = 40
NB = N // BF


def _fused_kernel(x_ref, t_ref, ltw_ref, ltb_ref, lw_ref, lb_ref,
                  W1_ref, b1_ref, w2_ref, b2_ref,
                  sel_ref, g_ref,
                  p_s, tn_s, f2f_s, vis_s):
    i = pl.program_id(0)

    @pl.when(i == 0)
    def _():
        t = t_ref[...]
        mu = jnp.mean(t, axis=1, keepdims=True)
        d = t - mu
        var = jnp.mean(d * d, axis=1, keepdims=True)
        tn_s[...] = d / jnp.sqrt(var + 1e-5) * ltw_ref[...] + ltb_ref[...]

    pooled = jnp.mean(x_ref[...], axis=1)
    mu = jnp.mean(pooled, axis=1, keepdims=True)
    d = pooled - mu
    var = jnp.mean(d * d, axis=1, keepdims=True)
    p_s[pl.ds(i * BF, BF), :] = (
        d / jnp.sqrt(var + 1e-5) * lw_ref[...] + lb_ref[...])

    @pl.when(i == NB - 1)
    def _():
        p = p_s[...]
        fused = jnp.concatenate(
            [jnp.broadcast_to(tn_s[...].astype(jnp.bfloat16), (N, EMBED)),
             p.astype(jnp.bfloat16)], axis=1)
        h = jnp.dot(fused, W1_ref[...].astype(jnp.bfloat16),
                    preferred_element_type=jnp.float32) + b1_ref[...]
        h = jnp.maximum(h, 0.0)
        logits = jax.lax.dot_general(
            w2_ref[...].astype(jnp.bfloat16), h.astype(jnp.bfloat16),
            (((1,), (1,)), ((), ())),
            preferred_element_type=jnp.float32) + b2_ref[...]
        gates = jax.nn.sigmoid(logits)
        g_ref[...] = gates

        nrm = jnp.sqrt(jnp.sum(p * p, axis=1, keepdims=True))
        pn = p / jnp.maximum(nrm, 1e-8)
        f2f_s[...] = jax.lax.dot_general(
            pn, pn, (((1,), (1,)), ((), ())),
            preferred_element_type=jnp.float32)

        idx = jax.lax.broadcasted_iota(jnp.int32, (1, N), 1)
        vis_s[...] = jnp.zeros((1, N), jnp.float32)
        sel_ref[...] = jnp.zeros((1, N), jnp.int32)

        def body(_, cnt):
            g = g_ref[...]
            v = vis_s[...]
            masked = jnp.where(v > 0.0, -jnp.inf, g)
            m = jnp.max(masked)
            cur = jnp.min(jnp.where(masked == m, idx, N))
            active = (m >= F2T_THRD) & (cnt < MAX_FRAME_NUM)
            row = f2f_s[pl.ds(cur, 1), :]
            nv = jnp.where((row > F2F_THRD) | (idx == cur), 1.0, v)
            vis_s[...] = jnp.where(active, nv, v)
            sel_ref[...] = jnp.where(active & (idx == cur), 1, sel_ref[...])
            return cnt + active.astype(jnp.int32)

        jax.lax.fori_loop(0, MAX_FRAME_NUM, body, jnp.int32(0))


def kernel(image_features, text_features, ln_text_w, ln_text_b,
           ln_local_w, ln_local_b, W1, b1, W2, b2):
    sel, gates = pl.pallas_call(
        _fused_kernel,
        grid=(NB,),
        in_specs=[
            pl.BlockSpec((BF, NP, EMBED), lambda i: (i, 0, 0)),
            pl.BlockSpec((1, EMBED), lambda i: (0, 0)),
            pl.BlockSpec((1, EMBED), lambda i: (0, 0)),
            pl.BlockSpec((1, EMBED), lambda i: (0, 0)),
            pl.BlockSpec((1, EMBED), lambda i: (0, 0)),
            pl.BlockSpec((1, EMBED), lambda i: (0, 0)),
            pl.BlockSpec((EMBED * 2, HID), lambda i: (0, 0)),
            pl.BlockSpec((1, HID), lambda i: (0, 0)),
            pl.BlockSpec((1, HID), lambda i: (0, 0)),
            pl.BlockSpec((1, 1), lambda i: (0, 0)),
        ],
        out_specs=[
            pl.BlockSpec((1, N), lambda i: (0, 0)),
            pl.BlockSpec((1, N), lambda i: (0, 0)),
        ],
        out_shape=[
            jax.ShapeDtypeStruct((1, N), jnp.int32),
            jax.ShapeDtypeStruct((1, N), jnp.float32),
        ],
        scratch_shapes=[
            pltpu.VMEM((N, EMBED), jnp.float32),
            pltpu.VMEM((1, EMBED), jnp.float32),
            pltpu.VMEM((N, N), jnp.float32),
            pltpu.VMEM((1, N), jnp.float32),
        ],
    )(image_features, text_features,
      ln_text_w.reshape(1, EMBED), ln_text_b.reshape(1, EMBED),
      ln_local_w.reshape(1, EMBED), ln_local_b.reshape(1, EMBED),
      W1, b1.reshape(1, HID), W2.reshape(1, HID), b2.reshape(1, 1))

    return (sel[0], gates[0])
